# Initial kernel scaffold; baseline (speedup 1.0000x reference)
import jax
import jax.numpy as jnp
from jax.experimental import pallas as pl


def kernel(sample, entity_embedding, relation_embedding):
    raise NotImplementedError("write your pallas kernel here")


# speedup vs baseline: 3.4735x; 3.4735x over previous
import jax
import jax.numpy as jnp
from jax import lax
from jax.experimental import pallas as pl
from jax.experimental.pallas import tpu as pltpu
from jax.experimental.pallas import tpu_sc as plsc

_GAMMA = 12.0
_NE = 1000
_NR = 1000
_D = 128
_B = 16384
_NS = 16
_DG = 8
_SG = 4
_DW = _D // _DG
_BS = _B // _SG
_ST = _BS // _DG
_L = 16


def _body(sample_hbm, ent_hbm, rel_hbm, out_hbm,
          samp_v, ent_v, rel_v, acc_v, sum_v, shared):
    c = lax.axis_index("c")
    s = lax.axis_index("s")
    dg = s % _DG
    sgl = s // _DG
    g = c * (_NS // _DG) + sgl

    pltpu.sync_copy(sample_hbm.at[pl.ds(g * _BS * 3, _BS * 3)], samp_v)
    pltpu.sync_copy(ent_hbm.at[pl.ds(dg * _DW * _NE, _DW * _NE)], ent_v)
    pltpu.sync_copy(rel_hbm.at[pl.ds(dg * _DW * _NR, _DW * _NR)], rel_v)

    iota3 = lax.iota(jnp.int32, _L) * 3

    def step(sv, carry):
        base3 = sv * (_L * 3)
        i0 = iota3 + base3
        hv = plsc.load_gather(samp_v, [i0])
        rv = plsc.load_gather(samp_v, [i0 + 1])
        tv = plsc.load_gather(samp_v, [i0 + 2])
        acc = jnp.zeros((_L,), jnp.float32)
        for d in range(_DW):
            e = plsc.load_gather(ent_v, [hv + d * _NE])
            r = plsc.load_gather(rel_v, [rv + d * _NR])
            t = plsc.load_gather(ent_v, [tv + d * _NE])
            acc = acc + jnp.abs(e + r - t)
        acc_v[pl.ds(sv * _L, _L)] = acc
        return carry

    lax.fori_loop(0, _BS // _L, step, 0)

    pltpu.sync_copy(acc_v, shared.at[sgl * _DG + dg])
    plsc.subcore_barrier()

    for row in range(_DG):
        pltpu.sync_copy(shared.at[sgl * _DG + row, pl.ds(dg * _ST, _ST)],
                        sum_v.at[pl.ds(row * _ST, _ST)])

    def red(k, carry):
        base = k * _L
        acc = sum_v[pl.ds(base, _L)]
        for row in range(1, _DG):
            acc = acc + sum_v[pl.ds(base + row * _ST, _L)]
        acc_v[pl.ds(base, _L)] = jnp.float32(_GAMMA) - acc
        return carry

    lax.fori_loop(0, _ST // _L, red, 0)

    pltpu.sync_copy(acc_v.at[pl.ds(0, _ST)],
                    out_hbm.at[pl.ds(g * _BS + dg * _ST, _ST)])


def kernel(sample, entity_embedding, relation_embedding):
    mesh = plsc.VectorSubcoreMesh(core_axis_name="c", subcore_axis_name="s")
    call = pl.kernel(
        _body,
        out_type=jax.ShapeDtypeStruct((_B,), jnp.float32),
        mesh=mesh,
        compiler_params=pltpu.CompilerParams(needs_layout_passes=False),
        scratch_types=[
            pltpu.VMEM((_BS * 3,), jnp.int32),
            pltpu.VMEM((_DW * _NE,), jnp.float32),
            pltpu.VMEM((_DW * _NR,), jnp.float32),
            pltpu.VMEM((_BS,), jnp.float32),
            pltpu.VMEM((_BS,), jnp.float32),
            pltpu.VMEM_SHARED((_NS, _BS), jnp.float32),
        ],
    )
    ent_t = entity_embedding.T.reshape(-1)
    rel_t = relation_embedding.T.reshape(-1)
    score = call(sample.reshape(-1), ent_t, rel_t)
    return score.reshape(_B, 1)
